# initial kernel scaffold (unmeasured)
import jax
import jax.numpy as jnp
from jax import lax
from jax.experimental import pallas as pl
from jax.experimental.pallas import tpu as pltpu

N_DEV = 4
SQ = 512
D = 1024
HEADS = 8
DH = 128
SCALE = 0.08838834764831843


def kernel(x, Wq, Wo, Wk, Wv):
    def body(x_ref, wq_ref, wo_ref, wk_ref, wv_ref, out_ref,
             xall_ref, part_ref, rs_recv_ref, rs_acc_ref,
             ag_send_sems, ag_recv_sems, rs_send_sems, rs_recv_sems):
        my = lax.axis_index("i")
        left = lax.rem(my + N_DEV - 1, N_DEV)
        right = lax.rem(my + 1, N_DEV)

        barrier = pltpu.get_barrier_semaphore()
        for nbr in (left, right):
            pl.semaphore_signal(barrier, inc=1, device_id=(nbr,),
                                device_id_type=pl.DeviceIdType.MESH)
        pl.semaphore_wait(barrier, 2)

        xall_ref[pl.ds(my, 1)] = x_ref[...]

        for h in range(N_DEV - 1):
            src_idx = lax.rem(my + N_DEV - h, N_DEV)
            rdma = pltpu.make_async_remote_copy(
                src_ref=xall_ref.at[pl.ds(src_idx, 1)],
                dst_ref=xall_ref.at[pl.ds(src_idx, 1)],
                send_sem=ag_send_sems.at[h],
                recv_sem=ag_recv_sems.at[h],
                device_id=(right,),
                device_id_type=pl.DeviceIdType.MESH,
            )
            rdma.start()
            rdma.wait()

        def attn_partial(xb):
            q = jnp.dot(xb, wq_ref[...], preferred_element_type=jnp.float32)
            k = jnp.dot(xb, wk_ref[...], preferred_element_type=jnp.float32)
            v = jnp.dot(xb, wv_ref[...], preferred_element_type=jnp.float32)
            outs = []
            for hh in range(HEADS):
                sl = slice(hh * DH, (hh + 1) * DH)
                s = jnp.dot(q[:, sl], k[:, sl].T,
                            preferred_element_type=jnp.float32) * SCALE
                m = jnp.max(s, axis=1, keepdims=True)
                p = jnp.exp(s - m)
                lsum = jnp.sum(p, axis=1, keepdims=True)
                outs.append(jnp.dot(p, v[:, sl],
                                    preferred_element_type=jnp.float32) / lsum)
            o_all = jnp.concatenate(outs, axis=1)
            return jnp.dot(o_all, wo_ref[...], preferred_element_type=jnp.float32)

        for b in range(N_DEV):
            part_ref[b] = attn_partial(xall_ref[b])

        for s in range(N_DEV - 1):
            c = lax.rem(my + 2 * N_DEV - 1 - s, N_DEV)
            mine = part_ref[pl.ds(c, 1)][0]
            if s == 0:
                rs_acc_ref[...] = mine
            else:
                rs_acc_ref[...] = rs_recv_ref[s - 1] + mine
            rdma = pltpu.make_async_remote_copy(
                src_ref=rs_acc_ref,
                dst_ref=rs_recv_ref.at[s],
                send_sem=rs_send_sems.at[s],
                recv_sem=rs_recv_sems.at[s],
                device_id=(right,),
                device_id_type=pl.DeviceIdType.MESH,
            )
            rdma.start()
            rdma.wait()

        out_ref[...] = rs_recv_ref[pl.ds(N_DEV - 2, 1)] + part_ref[pl.ds(my, 1)]

    return pl.pallas_call(
        body,
        out_shape=jax.ShapeDtypeStruct((1, SQ, D), jnp.float32),
        in_specs=[pl.BlockSpec(memory_space=pltpu.VMEM)] * 5,
        out_specs=pl.BlockSpec(memory_space=pltpu.VMEM),
        scratch_shapes=[
            pltpu.VMEM((N_DEV, SQ, D), jnp.float32),
            pltpu.VMEM((N_DEV, SQ, D), jnp.float32),
            pltpu.VMEM((N_DEV - 1, SQ, D), jnp.float32),
            pltpu.VMEM((SQ, D), jnp.float32),
            pltpu.SemaphoreType.DMA((N_DEV - 1,)),
            pltpu.SemaphoreType.DMA((N_DEV - 1,)),
            pltpu.SemaphoreType.DMA((N_DEV - 1,)),
            pltpu.SemaphoreType.DMA((N_DEV - 1,)),
        ],
        compiler_params=pltpu.CompilerParams(collective_id=0),
    )(x, Wq, Wo, Wk, Wv)


# baseline (device time: 189269 ns/iter reference)
import jax
import jax.numpy as jnp
from jax import lax
from jax.experimental import pallas as pl
from jax.experimental.pallas import tpu as pltpu

N_DEV = 4
SQ = 512
D = 1024
HEADS = 8
DH = 128
SCALE = 0.08838834764831843


def kernel(x, Wq, Wo, Wk, Wv):
    def body(x_ref, wq_ref, wo_ref, wk_ref, wv_ref, out_ref,
             xall_ref, part_ref, rs_recv_ref, rs_acc_ref,
             ag_send_sems, ag_recv_sems, rs_send_sems, rs_recv_sems):
        my = lax.axis_index("i")
        left = lax.rem(my + N_DEV - 1, N_DEV)
        right = lax.rem(my + 1, N_DEV)

        barrier = pltpu.get_barrier_semaphore()
        for nbr in (left, right):
            pl.semaphore_signal(barrier, inc=1, device_id=(nbr,),
                                device_id_type=pl.DeviceIdType.MESH)
        pl.semaphore_wait(barrier, 2)

        xall_ref[pl.ds(my, 1)] = x_ref[...]

        for h in range(N_DEV - 1):
            src_idx = lax.rem(my + N_DEV - h, N_DEV)
            rdma = pltpu.make_async_remote_copy(
                src_ref=xall_ref.at[pl.ds(src_idx, 1)],
                dst_ref=xall_ref.at[pl.ds(src_idx, 1)],
                send_sem=ag_send_sems.at[h],
                recv_sem=ag_recv_sems.at[h],
                device_id=(right,),
                device_id_type=pl.DeviceIdType.MESH,
            )
            rdma.start()
            rdma.wait()

        def attn_partial(xb):
            q = jnp.dot(xb, wq_ref[...], preferred_element_type=jnp.float32)
            k = jnp.dot(xb, wk_ref[...], preferred_element_type=jnp.float32)
            v = jnp.dot(xb, wv_ref[...], preferred_element_type=jnp.float32)
            outs = []
            for hh in range(HEADS):
                sl = slice(hh * DH, (hh + 1) * DH)
                s = jnp.dot(q[:, sl], k[:, sl].T,
                            preferred_element_type=jnp.float32) * SCALE
                m = jnp.max(s, axis=1, keepdims=True)
                p = jnp.exp(s - m)
                lsum = jnp.sum(p, axis=1, keepdims=True)
                outs.append(jnp.dot(p, v[:, sl],
                                    preferred_element_type=jnp.float32) / lsum)
            o_all = jnp.concatenate(outs, axis=1)
            return jnp.dot(o_all, wo_ref[...], preferred_element_type=jnp.float32)

        for b in range(N_DEV):
            part_ref[b] = attn_partial(xall_ref[b])

        for s in range(N_DEV - 1):
            c = lax.rem(my + 2 * N_DEV - 1 - s, N_DEV)
            mine = part_ref[pl.ds(c, 1)][0]
            if s == 0:
                rs_acc_ref[...] = mine
            else:
                rs_acc_ref[...] = rs_recv_ref[s - 1] + mine
            rdma = pltpu.make_async_remote_copy(
                src_ref=rs_acc_ref,
                dst_ref=rs_recv_ref.at[s],
                send_sem=rs_send_sems.at[s],
                recv_sem=rs_recv_sems.at[s],
                device_id=(right,),
                device_id_type=pl.DeviceIdType.MESH,
            )
            rdma.start()
            rdma.wait()

        out_ref[...] = rs_recv_ref[pl.ds(N_DEV - 2, 1)] + part_ref[pl.ds(my, 1)]

    return pl.pallas_call(
        body,
        out_shape=jax.ShapeDtypeStruct((1, SQ, D), jnp.float32),
        in_specs=[pl.BlockSpec(memory_space=pltpu.VMEM)] * 5,
        out_specs=pl.BlockSpec(memory_space=pltpu.VMEM),
        scratch_shapes=[
            pltpu.VMEM((N_DEV, SQ, D), jnp.float32),
            pltpu.VMEM((N_DEV, SQ, D), jnp.float32),
            pltpu.VMEM((N_DEV - 1, SQ, D), jnp.float32),
            pltpu.VMEM((SQ, D), jnp.float32),
            pltpu.SemaphoreType.DMA((N_DEV - 1,)),
            pltpu.SemaphoreType.DMA((N_DEV - 1,)),
            pltpu.SemaphoreType.DMA((N_DEV - 1,)),
            pltpu.SemaphoreType.DMA((N_DEV - 1,)),
        ],
        compiler_params=pltpu.CompilerParams(
            collective_id=0, vmem_limit_bytes=100 * 1024 * 1024
        ),
    )(x, Wq, Wo, Wk, Wv)


# device time: 79052 ns/iter; 2.3942x vs baseline; 2.3942x over previous
import jax
import jax.numpy as jnp
from jax import lax
from jax.experimental import pallas as pl
from jax.experimental.pallas import tpu as pltpu

N_DEV = 4
SQ = 512
D = 1024
HEADS = 8
DH = 128
SCALE = 0.08838834764831843

C16 = jnp.bfloat16


def kernel(x, Wq, Wo, Wk, Wv):
    def body(x_ref, wq_ref, wo_ref, wk_ref, wv_ref, out_ref,
             xg_ref, pin_ref, pout_ref, w16_ref,
             x_send_sems, x_recv_sems, p_send_sems, p_recv_sems):
        my = lax.axis_index("i")
        peers = [lax.rem(my + k, N_DEV) for k in range(1, N_DEV)]

        barrier = pltpu.get_barrier_semaphore()
        for p in peers:
            pl.semaphore_signal(barrier, inc=1, device_id=(p,),
                                device_id_type=pl.DeviceIdType.MESH)
        pl.semaphore_wait(barrier, N_DEV - 1)

        xg_ref[pl.ds(my, 1)] = x_ref[...].astype(C16)

        sends = []
        for k in range(1, N_DEV):
            rdma = pltpu.make_async_remote_copy(
                src_ref=xg_ref.at[pl.ds(my, 1)],
                dst_ref=xg_ref.at[pl.ds(my, 1)],
                send_sem=x_send_sems.at[k - 1],
                recv_sem=x_recv_sems.at[3 - k],
                device_id=(peers[k - 1],),
                device_id_type=pl.DeviceIdType.MESH,
            )
            rdma.start()
            sends.append(rdma)

        w16_ref[0] = wq_ref[...].astype(C16)
        w16_ref[1] = wk_ref[...].astype(C16)
        w16_ref[2] = wv_ref[...].astype(C16)
        w16_ref[3] = wo_ref[...].astype(C16)

        def attn_partial(xb16):
            q = jnp.dot(xb16, w16_ref[0], preferred_element_type=jnp.float32)
            k = jnp.dot(xb16, w16_ref[1], preferred_element_type=jnp.float32)
            v = jnp.dot(xb16, w16_ref[2], preferred_element_type=jnp.float32)
            q16, k16, v16 = q.astype(C16), k.astype(C16), v.astype(C16)
            outs = []
            for hh in range(HEADS):
                sl = slice(hh * DH, (hh + 1) * DH)
                s = jnp.dot(q16[:, sl], k16[:, sl].T,
                            preferred_element_type=jnp.float32) * SCALE
                m = jnp.max(s, axis=1, keepdims=True)
                p = jnp.exp(s - m)
                lsum = jnp.sum(p, axis=1, keepdims=True)
                o = jnp.dot(p.astype(C16), v16[:, sl],
                            preferred_element_type=jnp.float32) / lsum
                outs.append(o.astype(C16))
            o_all = jnp.concatenate(outs, axis=1)
            return jnp.dot(o_all, w16_ref[3], preferred_element_type=jnp.float32)

        out_ref[...] = attn_partial(xg_ref[pl.ds(my, 1)][0])[None]

        def recv_x(o):
            b = lax.rem(my + o, N_DEV)
            rdma = pltpu.make_async_remote_copy(
                src_ref=xg_ref.at[pl.ds(b, 1)],
                dst_ref=xg_ref.at[pl.ds(b, 1)],
                send_sem=x_send_sems.at[0],
                recv_sem=x_recv_sems.at[o - 1],
                device_id=(my,),
                device_id_type=pl.DeviceIdType.MESH,
            )
            rdma.wait_recv()
            return b

        for o in (1, 3, 2):
            b = recv_x(o)
            pout_ref[o - 1] = attn_partial(xg_ref[pl.ds(b, 1)][0]).astype(C16)
            rdma = pltpu.make_async_remote_copy(
                src_ref=pout_ref.at[o - 1],
                dst_ref=pin_ref.at[3 - o],
                send_sem=p_send_sems.at[o - 1],
                recv_sem=p_recv_sems.at[3 - o],
                device_id=(b,),
                device_id_type=pl.DeviceIdType.MESH,
            )
            rdma.start()
            sends.append(rdma)

        for j in range(N_DEV - 1):
            rdma = pltpu.make_async_remote_copy(
                src_ref=pout_ref.at[j],
                dst_ref=pin_ref.at[j],
                send_sem=p_send_sems.at[0],
                recv_sem=p_recv_sems.at[j],
                device_id=(my,),
                device_id_type=pl.DeviceIdType.MESH,
            )
            rdma.wait_recv()
        acc = (pin_ref[0].astype(jnp.float32) + pin_ref[1].astype(jnp.float32)
               + pin_ref[2].astype(jnp.float32))
        out_ref[...] = out_ref[...] + acc[None]

        for rdma in sends:
            rdma.wait_send()

    return pl.pallas_call(
        body,
        out_shape=jax.ShapeDtypeStruct((1, SQ, D), jnp.float32),
        in_specs=[pl.BlockSpec(memory_space=pltpu.VMEM)] * 5,
        out_specs=pl.BlockSpec(memory_space=pltpu.VMEM),
        scratch_shapes=[
            pltpu.VMEM((N_DEV, SQ, D), C16),
            pltpu.VMEM((N_DEV - 1, SQ, D), C16),
            pltpu.VMEM((N_DEV - 1, SQ, D), C16),
            pltpu.VMEM((4, D, D), C16),
            pltpu.SemaphoreType.DMA((N_DEV - 1,)),
            pltpu.SemaphoreType.DMA((N_DEV - 1,)),
            pltpu.SemaphoreType.DMA((N_DEV - 1,)),
            pltpu.SemaphoreType.DMA((N_DEV - 1,)),
        ],
        compiler_params=pltpu.CompilerParams(
            collective_id=0, vmem_limit_bytes=100 * 1024 * 1024
        ),
    )(x, Wq, Wo, Wk, Wv)


# device time: 67047 ns/iter; 2.8229x vs baseline; 1.1791x over previous
import jax
import jax.numpy as jnp
from jax import lax
from jax.experimental import pallas as pl
from jax.experimental.pallas import tpu as pltpu

N_DEV = 4
SQ = 512
D = 1024
HEADS = 8
DH = 128
SCALE = 0.08838834764831843

C16 = jnp.bfloat16


def kernel(x, Wq, Wo, Wk, Wv):
    def body(x_ref, wq_ref, wo_ref, wk_ref, wv_ref, out_ref,
             xg_ref, pin_ref, pout_ref, w16_ref, wo16_ref,
             x_send_sems, x_recv_sems, p_send_sems, p_recv_sems):
        my = lax.axis_index("i")
        peers = [lax.rem(my + k, N_DEV) for k in range(1, N_DEV)]

        barrier = pltpu.get_barrier_semaphore()
        for p in peers:
            pl.semaphore_signal(barrier, inc=1, device_id=(p,),
                                device_id_type=pl.DeviceIdType.MESH)
        pl.semaphore_wait(barrier, N_DEV - 1)

        xg_ref[pl.ds(my, 1)] = x_ref[...].astype(C16)

        def send_x(k):
            rdma = pltpu.make_async_remote_copy(
                src_ref=xg_ref.at[pl.ds(my, 1)],
                dst_ref=xg_ref.at[pl.ds(my, 1)],
                send_sem=x_send_sems.at[k - 1],
                recv_sem=x_recv_sems.at[3 - k],
                device_id=(peers[k - 1],),
                device_id_type=pl.DeviceIdType.MESH,
            )
            rdma.start()
            return rdma

        sends = [send_x(1), send_x(3)]

        w16_ref[:, pl.ds(0, D)] = wq_ref[...].astype(C16)
        w16_ref[:, pl.ds(D, D)] = wk_ref[...].astype(C16)
        w16_ref[:, pl.ds(2 * D, D)] = wv_ref[...].astype(C16)
        wo16_ref[...] = wo_ref[...].astype(C16)

        def attn_partial(xb16):
            qkv = jnp.dot(xb16, w16_ref[...], preferred_element_type=jnp.float32)
            qkv16 = qkv.astype(C16)
            outs = []
            for hh in range(HEADS):
                q16 = qkv16[:, hh * DH:(hh + 1) * DH]
                k16 = qkv16[:, D + hh * DH:D + (hh + 1) * DH]
                v16 = qkv16[:, 2 * D + hh * DH:2 * D + (hh + 1) * DH]
                s = jnp.dot(q16, k16.T,
                            preferred_element_type=jnp.float32) * SCALE
                p = jnp.exp(s)
                lsum = jnp.sum(p, axis=1, keepdims=True)
                o = jnp.dot(p.astype(C16), v16,
                            preferred_element_type=jnp.float32) / lsum
                outs.append(o.astype(C16))
            o_all = jnp.concatenate(outs, axis=1)
            return jnp.dot(o_all, wo16_ref[...], preferred_element_type=jnp.float32)

        out_ref[...] = attn_partial(xg_ref[pl.ds(my, 1)][0])[None]

        sends[0].wait_send()
        sends[1].wait_send()
        sends = [send_x(2)]

        def recv_x(o):
            b = lax.rem(my + o, N_DEV)
            rdma = pltpu.make_async_remote_copy(
                src_ref=xg_ref.at[pl.ds(b, 1)],
                dst_ref=xg_ref.at[pl.ds(b, 1)],
                send_sem=x_send_sems.at[0],
                recv_sem=x_recv_sems.at[o - 1],
                device_id=(my,),
                device_id_type=pl.DeviceIdType.MESH,
            )
            rdma.wait_recv()
            return b

        for o in (1, 3, 2):
            b = recv_x(o)
            pout_ref[o - 1] = attn_partial(xg_ref[pl.ds(b, 1)][0]).astype(C16)
            rdma = pltpu.make_async_remote_copy(
                src_ref=pout_ref.at[o - 1],
                dst_ref=pin_ref.at[3 - o],
                send_sem=p_send_sems.at[o - 1],
                recv_sem=p_recv_sems.at[3 - o],
                device_id=(b,),
                device_id_type=pl.DeviceIdType.MESH,
            )
            rdma.start()
            sends.append(rdma)

        for j in range(N_DEV - 1):
            rdma = pltpu.make_async_remote_copy(
                src_ref=pout_ref.at[j],
                dst_ref=pin_ref.at[j],
                send_sem=p_send_sems.at[0],
                recv_sem=p_recv_sems.at[j],
                device_id=(my,),
                device_id_type=pl.DeviceIdType.MESH,
            )
            rdma.wait_recv()
        acc = (pin_ref[0].astype(jnp.float32) + pin_ref[1].astype(jnp.float32)
               + pin_ref[2].astype(jnp.float32))
        out_ref[...] = out_ref[...] + acc[None]

        for rdma in sends:
            rdma.wait_send()

    return pl.pallas_call(
        body,
        out_shape=jax.ShapeDtypeStruct((1, SQ, D), jnp.float32),
        in_specs=[pl.BlockSpec(memory_space=pltpu.VMEM)] * 5,
        out_specs=pl.BlockSpec(memory_space=pltpu.VMEM),
        scratch_shapes=[
            pltpu.VMEM((N_DEV, SQ, D), C16),
            pltpu.VMEM((N_DEV - 1, SQ, D), C16),
            pltpu.VMEM((N_DEV - 1, SQ, D), C16),
            pltpu.VMEM((D, 3 * D), C16),
            pltpu.VMEM((D, D), C16),
            pltpu.SemaphoreType.DMA((N_DEV - 1,)),
            pltpu.SemaphoreType.DMA((N_DEV - 1,)),
            pltpu.SemaphoreType.DMA((N_DEV - 1,)),
            pltpu.SemaphoreType.DMA((N_DEV - 1,)),
        ],
        compiler_params=pltpu.CompilerParams(
            collective_id=0, vmem_limit_bytes=100 * 1024 * 1024
        ),
    )(x, Wq, Wo, Wk, Wv)


# device time: 66800 ns/iter; 2.8334x vs baseline; 1.0037x over previous
import jax
import jax.numpy as jnp
from jax import lax
from jax.experimental import pallas as pl
from jax.experimental.pallas import tpu as pltpu

N_DEV = 4
SQ = 512
D = 1024
HEADS = 8
DH = 128
SCALE = 0.08838834764831843

C16 = jnp.bfloat16


def kernel(x, Wq, Wo, Wk, Wv):
    def body(x_ref, wq_ref, wo_ref, wk_ref, wv_ref, out_ref,
             xg_ref, pin_ref, pout_ref, w16_ref, wo16_ref,
             x_send_sems, x_recv_sems, p_send_sems, p_recv_sems):
        my = lax.axis_index("i")
        peers = [lax.rem(my + k, N_DEV) for k in range(1, N_DEV)]

        barrier = pltpu.get_barrier_semaphore()
        for p in peers:
            pl.semaphore_signal(barrier, inc=1, device_id=(p,),
                                device_id_type=pl.DeviceIdType.MESH)
        pl.semaphore_wait(barrier, N_DEV - 1)

        xg_ref[pl.ds(my, 1)] = x_ref[...].astype(C16)

        def send_x(k):
            rdma = pltpu.make_async_remote_copy(
                src_ref=xg_ref.at[pl.ds(my, 1)],
                dst_ref=xg_ref.at[pl.ds(my, 1)],
                send_sem=x_send_sems.at[k - 1],
                recv_sem=x_recv_sems.at[3 - k],
                device_id=(peers[k - 1],),
                device_id_type=pl.DeviceIdType.MESH,
            )
            rdma.start()
            return rdma

        sends = [send_x(1), send_x(3)]

        w16_ref[:, pl.ds(0, D)] = wq_ref[...].astype(C16)
        w16_ref[:, pl.ds(D, D)] = wk_ref[...].astype(C16)
        w16_ref[:, pl.ds(2 * D, D)] = wv_ref[...].astype(C16)
        wo16_ref[...] = wo_ref[...].astype(C16)

        def attn_partial(xb16, out_dtype):
            qkv16 = jnp.dot(xb16, w16_ref[...],
                            preferred_element_type=jnp.float32).astype(C16)
            outs = []
            for hh in range(HEADS):
                q16 = qkv16[:, hh * DH:(hh + 1) * DH]
                k16 = qkv16[:, D + hh * DH:D + (hh + 1) * DH]
                v16 = qkv16[:, 2 * D + hh * DH:2 * D + (hh + 1) * DH]
                s = jnp.dot(q16, k16.T,
                            preferred_element_type=jnp.float32) * SCALE
                p16 = jnp.exp(s).astype(C16)
                lsum = jnp.sum(p16, axis=1, keepdims=True,
                               dtype=jnp.float32)
                o = jnp.dot(p16, v16, preferred_element_type=jnp.float32)
                outs.append((o / lsum).astype(C16))
            o_all = jnp.concatenate(outs, axis=1)
            res = jnp.dot(o_all, wo16_ref[...],
                          preferred_element_type=jnp.float32)
            return res if out_dtype == jnp.float32 else res.astype(out_dtype)

        out_ref[...] = attn_partial(xg_ref[pl.ds(my, 1)][0], jnp.float32)[None]

        sends[0].wait_send()
        sends[1].wait_send()
        sends = [send_x(2)]

        def recv_x(o):
            b = lax.rem(my + o, N_DEV)
            rdma = pltpu.make_async_remote_copy(
                src_ref=xg_ref.at[pl.ds(b, 1)],
                dst_ref=xg_ref.at[pl.ds(b, 1)],
                send_sem=x_send_sems.at[0],
                recv_sem=x_recv_sems.at[o - 1],
                device_id=(my,),
                device_id_type=pl.DeviceIdType.MESH,
            )
            rdma.wait_recv()
            return b

        for o in (1, 3, 2):
            b = recv_x(o)
            pout_ref[o - 1] = attn_partial(xg_ref[pl.ds(b, 1)][0], C16)
            rdma = pltpu.make_async_remote_copy(
                src_ref=pout_ref.at[o - 1],
                dst_ref=pin_ref.at[3 - o],
                send_sem=p_send_sems.at[o - 1],
                recv_sem=p_recv_sems.at[3 - o],
                device_id=(b,),
                device_id_type=pl.DeviceIdType.MESH,
            )
            rdma.start()
            sends.append(rdma)

        for j in (2, 0, 1):
            rdma = pltpu.make_async_remote_copy(
                src_ref=pout_ref.at[j],
                dst_ref=pin_ref.at[j],
                send_sem=p_send_sems.at[0],
                recv_sem=p_recv_sems.at[j],
                device_id=(my,),
                device_id_type=pl.DeviceIdType.MESH,
            )
            rdma.wait_recv()
            out_ref[...] = out_ref[...] + pin_ref[j].astype(jnp.float32)[None]

        for rdma in sends:
            rdma.wait_send()

    return pl.pallas_call(
        body,
        out_shape=jax.ShapeDtypeStruct((1, SQ, D), jnp.float32),
        in_specs=[pl.BlockSpec(memory_space=pltpu.VMEM)] * 5,
        out_specs=pl.BlockSpec(memory_space=pltpu.VMEM),
        scratch_shapes=[
            pltpu.VMEM((N_DEV, SQ, D), C16),
            pltpu.VMEM((N_DEV - 1, SQ, D), C16),
            pltpu.VMEM((N_DEV - 1, SQ, D), C16),
            pltpu.VMEM((D, 3 * D), C16),
            pltpu.VMEM((D, D), C16),
            pltpu.SemaphoreType.DMA((N_DEV - 1,)),
            pltpu.SemaphoreType.DMA((N_DEV - 1,)),
            pltpu.SemaphoreType.DMA((N_DEV - 1,)),
            pltpu.SemaphoreType.DMA((N_DEV - 1,)),
        ],
        compiler_params=pltpu.CompilerParams(
            collective_id=0, vmem_limit_bytes=100 * 1024 * 1024
        ),
    )(x, Wq, Wo, Wk, Wv)


# device time: 66041 ns/iter; 2.8659x vs baseline; 1.0115x over previous
import jax
import jax.numpy as jnp
from jax import lax
from jax.experimental import pallas as pl
from jax.experimental.pallas import tpu as pltpu

N_DEV = 4
SQ = 512
D = 1024
HEADS = 8
DH = 128
SCALE = 0.08838834764831843

C16 = jnp.bfloat16


def kernel(x, Wq, Wo, Wk, Wv):
    def body(x_ref, wq_ref, wo_ref, wk_ref, wv_ref, out_ref,
             xg_ref, pin_ref, pout_ref, w16_ref, wo16_ref,
             x_send_sems, x_recv_sems, p_send_sems, p_recv_sems):
        my = lax.axis_index("i")
        peers = [lax.rem(my + k, N_DEV) for k in range(1, N_DEV)]

        barrier = pltpu.get_barrier_semaphore()
        for p in peers:
            pl.semaphore_signal(barrier, inc=1, device_id=(p,),
                                device_id_type=pl.DeviceIdType.MESH)
        pl.semaphore_wait(barrier, N_DEV - 1)

        xg_ref[pl.ds(my, 1)] = x_ref[...].astype(C16)

        def send_x(k):
            rdma = pltpu.make_async_remote_copy(
                src_ref=xg_ref.at[pl.ds(my, 1)],
                dst_ref=xg_ref.at[pl.ds(my, 1)],
                send_sem=x_send_sems.at[k - 1],
                recv_sem=x_recv_sems.at[3 - k],
                device_id=(peers[k - 1],),
                device_id_type=pl.DeviceIdType.MESH,
            )
            rdma.start()
            return rdma

        sends = [send_x(1), send_x(3)]

        w16_ref[:, pl.ds(0, D)] = wq_ref[...].astype(C16)
        w16_ref[:, pl.ds(D, D)] = wk_ref[...].astype(C16)
        w16_ref[:, pl.ds(2 * D, D)] = wv_ref[...].astype(C16)
        wo16_ref[...] = wo_ref[...].astype(C16)

        def attn_heads(xb16):
            qkv16 = jnp.dot(xb16, w16_ref[...],
                            preferred_element_type=jnp.float32).astype(C16)
            outs = []
            for hh in range(HEADS):
                q16 = qkv16[:, hh * DH:(hh + 1) * DH]
                k16 = qkv16[:, D + hh * DH:D + (hh + 1) * DH]
                v16 = qkv16[:, 2 * D + hh * DH:2 * D + (hh + 1) * DH]
                s = jnp.dot(q16, k16.T,
                            preferred_element_type=jnp.float32) * SCALE
                p16 = jnp.exp(s).astype(C16)
                lsum = jnp.sum(p16, axis=1, keepdims=True,
                               dtype=jnp.float32)
                o = jnp.dot(p16, v16, preferred_element_type=jnp.float32)
                outs.append((o / lsum).astype(C16))
            return jnp.concatenate(outs, axis=1)

        def attn_partial(xb16, out_dtype):
            res = jnp.dot(attn_heads(xb16), wo16_ref[...],
                          preferred_element_type=jnp.float32)
            return res if out_dtype == jnp.float32 else res.astype(out_dtype)

        out_ref[...] = attn_partial(xg_ref[pl.ds(my, 1)][0], jnp.float32)[None]

        sends[0].wait_send()
        sends[1].wait_send()
        sends = [send_x(2)]

        def recv_x(o):
            b = lax.rem(my + o, N_DEV)
            rdma = pltpu.make_async_remote_copy(
                src_ref=xg_ref.at[pl.ds(b, 1)],
                dst_ref=xg_ref.at[pl.ds(b, 1)],
                send_sem=x_send_sems.at[0],
                recv_sem=x_recv_sems.at[o - 1],
                device_id=(my,),
                device_id_type=pl.DeviceIdType.MESH,
            )
            rdma.wait_recv()
            return b

        HALF = SQ // 2
        for o in (1, 3, 2):
            b = recv_x(o)
            o_all = attn_heads(xg_ref[pl.ds(b, 1)][0])
            for i in range(2):
                rows = pl.ds(i * HALF, HALF)
                pout_ref[o - 1, rows] = jnp.dot(
                    o_all[i * HALF:(i + 1) * HALF], wo16_ref[...],
                    preferred_element_type=jnp.float32).astype(C16)
                rdma = pltpu.make_async_remote_copy(
                    src_ref=pout_ref.at[o - 1, rows],
                    dst_ref=pin_ref.at[3 - o, rows],
                    send_sem=p_send_sems.at[2 * (o - 1) + i],
                    recv_sem=p_recv_sems.at[2 * (3 - o) + i],
                    device_id=(b,),
                    device_id_type=pl.DeviceIdType.MESH,
                )
                rdma.start()
                sends.append(rdma)

        for j in (2, 0, 1):
            for i in range(2):
                rdma = pltpu.make_async_remote_copy(
                    src_ref=pout_ref.at[j, pl.ds(i * HALF, HALF)],
                    dst_ref=pin_ref.at[j, pl.ds(i * HALF, HALF)],
                    send_sem=p_send_sems.at[0],
                    recv_sem=p_recv_sems.at[2 * j + i],
                    device_id=(my,),
                    device_id_type=pl.DeviceIdType.MESH,
                )
                rdma.wait_recv()
            out_ref[...] = out_ref[...] + pin_ref[j].astype(jnp.float32)[None]

        for rdma in sends:
            rdma.wait_send()

    return pl.pallas_call(
        body,
        out_shape=jax.ShapeDtypeStruct((1, SQ, D), jnp.float32),
        in_specs=[pl.BlockSpec(memory_space=pltpu.VMEM)] * 5,
        out_specs=pl.BlockSpec(memory_space=pltpu.VMEM),
        scratch_shapes=[
            pltpu.VMEM((N_DEV, SQ, D), C16),
            pltpu.VMEM((N_DEV - 1, SQ, D), C16),
            pltpu.VMEM((N_DEV - 1, SQ, D), C16),
            pltpu.VMEM((D, 3 * D), C16),
            pltpu.VMEM((D, D), C16),
            pltpu.SemaphoreType.DMA((N_DEV - 1,)),
            pltpu.SemaphoreType.DMA((N_DEV - 1,)),
            pltpu.SemaphoreType.DMA((2 * (N_DEV - 1),)),
            pltpu.SemaphoreType.DMA((2 * (N_DEV - 1),)),
        ],
        compiler_params=pltpu.CompilerParams(
            collective_id=0, vmem_limit_bytes=100 * 1024 * 1024
        ),
    )(x, Wq, Wo, Wk, Wv)
